# baseline (speedup 1.0000x reference)
import functools

import jax
import jax.numpy as jnp
from jax import lax
from jax.experimental import pallas as pl
from jax.experimental.pallas import tpu as pltpu
from jax.experimental.pallas import tpu_sc as plsc

BATCH = 128
C = 32768
K = 25
L = 16
NVREG = C // L
NW = 32
ROWS_PER_W = BATCH // NW
NBINS = 256
NCH = 4
CH = C // NCH
CHV = CH // L


def _key_from_float(y):
    s = plsc.bitcast(y, jnp.int32)
    return s ^ (jnp.right_shift(s, 31) & jnp.int32(0x7FFFFFFF))


def _float_from_key(k):
    return plsc.bitcast(k ^ (jnp.right_shift(k, 31) & jnp.int32(0x7FFFFFFF)),
                        jnp.float32)


def _sc_body(x_hbm, w_hbm, out_hbm, w_v, row_v, cand_v, hist,
             sem_in0, sem_in1, sem_in2, sem_o0, sem_o1, sem_o2, sem_o3):
    wid = lax.axis_index("s") * 2 + lax.axis_index("c")
    lane = lax.iota(jnp.int32, L)
    ones = jnp.ones((L,), jnp.int32)
    sems_in = (sem_in0, sem_in1, sem_in2)
    sems_out = (sem_o0, sem_o1, sem_o2, sem_o3)

    pltpu.async_copy(x_hbm.at[wid * ROWS_PER_W, pl.ds(0, CH)],
                     row_v.at[pl.ds(0, CH)], sems_in[0])
    pltpu.sync_copy(w_hbm, w_v)

    def row_fn(r, _):
        row = wid * ROWS_PER_W + r

        def issue_in(ch):
            sem = sems_in[2] if ch == 2 else sems_in[ch % 2]
            return pltpu.async_copy(
                x_hbm.at[row, pl.ds(ch * CH, CH)],
                row_v.at[pl.ds(ch * CH, CH)],
                sem)

        nxt = issue_in(1)
        pltpu.make_async_copy(
            x_hbm.at[row, pl.ds(0, CH)],
            row_v.at[pl.ds(0, CH)],
            sems_in[0]).wait()

        with jax.named_scope("est"):
            @plsc.parallel_loop(0, 64, unroll=8,
                                carry=jnp.full((L,), jnp.int32(-0x80000000)))
            def mx(i, mx):
                sl = pl.ds(i * (CHV // 64) * L, L)
                return jnp.maximum(mx, _key_from_float(row_v[sl] * w_v[sl]))
            mx_sorted, _ = plsc.sort_key_val(mx, mx, descending=True)
            cand_v[pl.ds(0, L)] = mx_sorted
            tau_est = plsc.load_gather(cand_v,
                                       [jnp.full((L,), 8, jnp.int32)])

        ptr = jnp.zeros((L,), jnp.int32)
        with jax.named_scope("p1"):
            for ch in range(NCH):
                if ch > 0:
                    in_h = nxt
                    nxt = issue_in(ch + 1) if ch + 1 < NCH else None
                    in_h.wait()
                base = ch * CHV

                @plsc.parallel_loop(0, CHV, unroll=8, carry=ptr)
                def ptr(i, ptr, base=base):
                    sl = pl.ds((base + i) * L, L)
                    k = _key_from_float(row_v[sl] * w_v[sl])
                    row_v[sl] = plsc.bitcast(k, jnp.float32)
                    keep = k >= tau_est
                    ki = keep.astype(jnp.int32)
                    idx = ptr + plsc.cumsum(ki) - ki
                    plsc.store_scatter(cand_v, [idx],
                                       k ^ jnp.int32(-0x80000000), mask=keep)
                    return ptr + plsc.all_reduce_population_count(keep)
            plsc.store_scatter(
                cand_v, [jnp.minimum(ptr + lane, jnp.int32(C - 1))],
                jnp.zeros((L,), jnp.int32))
            n = jnp.max(ptr)

        def search(nv, limit, krem0, hb, p0_u, masked):
            p = jnp.zeros((L,), jnp.uint32) | p0_u
            krem_v = jnp.zeros((L,), jnp.int32) + krem0

            for bi in range(31, -1, -1):
                def active(p, krem_v, bi=bi):
                    patt = jnp.right_shift(p | jnp.uint32(1 << bi),
                                           jnp.uint32(bi))

                    def cb(i, c, patt=patt, bi=bi):
                        v = plsc.bitcast(cand_v[pl.ds(i * L, L)],
                                         jnp.uint32)
                        m = jnp.right_shift(v, jnp.uint32(bi)) == patt
                        if masked:
                            m = jnp.logical_and(m, (i * L + lane) < limit)
                        return c + plsc.all_reduce_population_count(m)
                    c = plsc.parallel_loop(
                        0, nv, unroll=4,
                        carry=jnp.zeros((L,), jnp.int32))(cb)
                    ge = c >= krem_v
                    return (jnp.where(ge, p | jnp.uint32(1 << bi), p),
                            jnp.where(ge, krem_v, krem_v - c))

                p, krem_v = lax.cond(bi <= hb, active,
                                     lambda p, krem_v: (p, krem_v),
                                     p, krem_v)
            return plsc.bitcast(p, jnp.int32) ^ jnp.int32(-0x80000000)

        def good_path(n, ptr):
            nv = (n + (L - 1)) // L

            @plsc.parallel_loop(0, nv, unroll=2,
                                carry=jnp.full((L,), jnp.int32(-0x80000000)))
            def mk(i, mk):
                kv = cand_v[pl.ds(i * L, L)] ^ jnp.int32(-0x80000000)
                return jnp.maximum(mk, kv)
            sbit = jnp.int32(-0x80000000)
            mx_b = jnp.max(mk) ^ sbit
            tau_b = jnp.max(tau_est) ^ sbit
            diff = mx_b ^ tau_b

            def hbody(j, hb):
                return jnp.where(
                    lax.shift_right_logical(diff, j) != 0, j, hb)
            hb = lax.fori_loop(0, 32, hbody, jnp.int32(-1))
            sh = jnp.int32(31) - jnp.clip(hb, 0, 31)
            low = lax.shift_right_logical(jnp.int32(-1), sh)
            p0 = mx_b & ~jnp.where(hb < 0, jnp.int32(0), low)
            return search(nv, ptr, jnp.int32(K),
                          hb, p0.astype(jnp.uint32), False)

        def fallback_path(n, ptr):
            @plsc.parallel_loop(0, NBINS, unroll=8)
            def _(b):
                hist[b] = jnp.zeros((L,), jnp.int32)

            @plsc.parallel_loop(0, NVREG, unroll=8)
            def _(i):
                k = plsc.bitcast(row_v[pl.ds(i * L, L)], jnp.int32)
                byte = ((jnp.right_shift(k, 24) & jnp.int32(0xFF))
                        ^ jnp.int32(0x80))
                plsc.addupdate_scatter(hist, [byte, lane], ones)

            @plsc.parallel_loop(
                0, NBINS, unroll=8,
                carry=(jnp.int32(0), jnp.int32(-1), jnp.int32(0)))
            def sb_out(j, carry):
                acc, bsel, above = carry
                b = jnp.int32(NBINS - 1) - j
                cnt = jnp.sum(hist[b])
                found_now = jnp.logical_and(bsel < 0, acc + cnt >= K)
                bsel = jnp.where(found_now, b, bsel)
                above = jnp.where(found_now, acc, above)
                return (acc + cnt, bsel, above)
            _, b1, above = sb_out
            krem = jnp.int32(K) - above
            b1 = b1 ^ jnp.int32(0x80)

            @plsc.parallel_loop(0, NVREG, unroll=4,
                                carry=jnp.zeros((L,), jnp.int32))
            def ptr2(i, ptr2):
                kv = plsc.bitcast(row_v[pl.ds(i * L, L)], jnp.int32)
                keep = (jnp.right_shift(kv, 24) & jnp.int32(0xFF)) == b1
                ki = keep.astype(jnp.int32)
                idx = ptr2 + plsc.cumsum(ki) - ki
                plsc.store_scatter(cand_v, [idx],
                                   kv ^ jnp.int32(-0x80000000), mask=keep)
                return ptr2 + plsc.all_reduce_population_count(keep)
            n2 = jnp.max(ptr2)
            nv2 = (n2 + (L - 1)) // L
            p0 = jnp.left_shift(b1 ^ jnp.int32(0x80), 24)
            return search(nv2, ptr2, krem,
                          jnp.int32(23), p0.astype(jnp.uint32), True)

        with jax.named_scope("sel"):
            good = jnp.logical_and(n >= K, n <= C - L)
            tau = lax.cond(good, good_path, fallback_path, n, ptr)

        out_h = []
        with jax.named_scope("p3"):
            for ch in range(NCH):
                if ch == 2:
                    out_h[0].wait()
                    row_n = jnp.minimum(row + 1, jnp.int32(BATCH - 1))
                    pltpu.async_copy(
                        x_hbm.at[row_n, pl.ds(0, CH)],
                        row_v.at[pl.ds(0, CH)],
                        sems_in[0])
                base = ch * CHV

                @plsc.parallel_loop(0, CHV, unroll=8)
                def _(i, base=base):
                    sl = pl.ds((base + i) * L, L)
                    kv = plsc.bitcast(row_v[sl], jnp.int32)
                    y = _float_from_key(kv)
                    row_v[sl] = jnp.where(kv >= tau, y, jnp.float32(0.0))

                out_h.append(pltpu.async_copy(
                    row_v.at[pl.ds(ch * CH, CH)],
                    out_hbm.at[row, pl.ds(ch * CH, CH)],
                    sems_out[ch]))
            for h in out_h[1:]:
                h.wait()
        return 0

    lax.fori_loop(0, ROWS_PER_W, row_fn, 0)
    pltpu.make_async_copy(
        x_hbm.at[0, pl.ds(0, CH)],
        row_v.at[pl.ds(0, CH)],
        sems_in[0]).wait()


@functools.partial(jax.jit)
def _sc_kfeature(x, w):
    mesh = plsc.VectorSubcoreMesh(core_axis_name="c", subcore_axis_name="s")
    f = functools.partial(
        pl.kernel,
        mesh=mesh,
        compiler_params=pltpu.CompilerParams(needs_layout_passes=False),
        out_type=jax.ShapeDtypeStruct((BATCH, C), jnp.float32),
        scratch_types=[
            pltpu.VMEM((C,), jnp.float32),
            pltpu.VMEM((C,), jnp.float32),
            pltpu.VMEM((C,), jnp.int32),
            pltpu.VMEM((NBINS, L), jnp.int32),
            pltpu.SemaphoreType.DMA,
            pltpu.SemaphoreType.DMA,
            pltpu.SemaphoreType.DMA,
            pltpu.SemaphoreType.DMA,
            pltpu.SemaphoreType.DMA,
            pltpu.SemaphoreType.DMA,
            pltpu.SemaphoreType.DMA,
        ],
    )(_sc_body)
    return f(x, w)


def kernel(x, w):
    return _sc_kfeature(x, w)


# speedup vs baseline: 1.0526x; 1.0526x over previous
import functools

import jax
import jax.numpy as jnp
from jax import lax
from jax.experimental import pallas as pl
from jax.experimental.pallas import tpu as pltpu
from jax.experimental.pallas import tpu_sc as plsc

BATCH = 128
C = 32768
K = 25
L = 16
NVREG = C // L
NW = 32
ROWS_PER_W = BATCH // NW
NBINS = 256
NCH = 4
CH = C // NCH
CHV = CH // L


def _key_from_float(y):
    s = plsc.bitcast(y, jnp.int32)
    return s ^ (jnp.right_shift(s, 31) & jnp.int32(0x7FFFFFFF))


def _float_from_key(k):
    return plsc.bitcast(k ^ (jnp.right_shift(k, 31) & jnp.int32(0x7FFFFFFF)),
                        jnp.float32)


def _sc_body(x_hbm, w_hbm, out_hbm, w_v, row_v, cand_v, hist,
             sem_in0, sem_in1, sem_in2, sem_o0, sem_o1, sem_o2, sem_o3):
    wid = lax.axis_index("s") * 2 + lax.axis_index("c")
    lane = lax.iota(jnp.int32, L)
    ones = jnp.ones((L,), jnp.int32)
    sems_in = (sem_in0, sem_in1, sem_in2)
    sems_out = (sem_o0, sem_o1, sem_o2, sem_o3)

    pltpu.async_copy(x_hbm.at[wid * ROWS_PER_W, pl.ds(0, CH)],
                     row_v.at[pl.ds(0, CH)], sems_in[0])
    pltpu.sync_copy(w_hbm, w_v)

    def row_fn(r, _):
        row = wid * ROWS_PER_W + r

        def issue_in(ch):
            sem = sems_in[2] if ch == 2 else sems_in[ch % 2]
            return pltpu.async_copy(
                x_hbm.at[row, pl.ds(ch * CH, CH)],
                row_v.at[pl.ds(ch * CH, CH)],
                sem)

        nxt = issue_in(1)
        pltpu.make_async_copy(
            x_hbm.at[row, pl.ds(0, CH)],
            row_v.at[pl.ds(0, CH)],
            sems_in[0]).wait()

        with jax.named_scope("est"):
            @plsc.parallel_loop(0, 64, unroll=8,
                                carry=jnp.full((L,), jnp.int32(-0x80000000)))
            def mx(i, mx):
                sl = pl.ds(i * (CHV // 64) * L, L)
                return jnp.maximum(mx, _key_from_float(row_v[sl] * w_v[sl]))
            mx_sorted, _ = plsc.sort_key_val(mx, mx, descending=True)
            cand_v[pl.ds(0, L)] = mx_sorted
            tau_est = plsc.load_gather(cand_v,
                                       [jnp.full((L,), 8, jnp.int32)])

        ptr = jnp.zeros((L,), jnp.int32)
        with jax.named_scope("p1"):
            for ch in range(NCH):
                if ch > 0:
                    in_h = nxt
                    nxt = issue_in(ch + 1) if ch + 1 < NCH else None
                    in_h.wait()
                base = ch * CHV

                @plsc.parallel_loop(0, CHV, unroll=8, carry=ptr)
                def ptr(i, ptr, base=base):
                    sl = pl.ds((base + i) * L, L)
                    k = _key_from_float(row_v[sl] * w_v[sl])
                    row_v[sl] = plsc.bitcast(k, jnp.float32)
                    keep = k >= tau_est
                    ki = keep.astype(jnp.int32)
                    idx = ptr + plsc.cumsum(ki) - ki
                    plsc.store_scatter(cand_v, [idx],
                                       k ^ jnp.int32(-0x80000000), mask=keep)
                    return ptr + plsc.all_reduce_population_count(keep)
            plsc.store_scatter(
                cand_v, [jnp.minimum(ptr + lane, jnp.int32(C - 1))],
                jnp.zeros((L,), jnp.int32))
            n = jnp.max(ptr)

        def search(nv, limit, krem0, hb, p0_u, masked):
            p = jnp.zeros((L,), jnp.uint32) | p0_u
            krem_v = jnp.zeros((L,), jnp.int32) + krem0

            for bi in range(31, -1, -1):
                def active(p, krem_v, bi=bi):
                    patt = jnp.right_shift(p | jnp.uint32(1 << bi),
                                           jnp.uint32(bi))

                    def cb(i, c, patt=patt, bi=bi):
                        v = plsc.bitcast(cand_v[pl.ds(i * L, L)],
                                         jnp.uint32)
                        m = jnp.right_shift(v, jnp.uint32(bi)) == patt
                        if masked:
                            m = jnp.logical_and(m, (i * L + lane) < limit)
                        return c + plsc.all_reduce_population_count(m)
                    c = plsc.parallel_loop(
                        0, nv, unroll=2,
                        carry=jnp.zeros((L,), jnp.int32))(cb)
                    ge = c >= krem_v
                    return (jnp.where(ge, p | jnp.uint32(1 << bi), p),
                            jnp.where(ge, krem_v, krem_v - c))

                p, krem_v = lax.cond(bi <= hb, active,
                                     lambda p, krem_v: (p, krem_v),
                                     p, krem_v)
            return plsc.bitcast(p, jnp.int32) ^ jnp.int32(-0x80000000)

        def good_path(n, ptr):
            nv = (n + (L - 1)) // L

            @plsc.parallel_loop(0, nv, unroll=2,
                                carry=jnp.full((L,), jnp.int32(-0x80000000)))
            def mk(i, mk):
                kv = cand_v[pl.ds(i * L, L)] ^ jnp.int32(-0x80000000)
                return jnp.maximum(mk, kv)
            sbit = jnp.int32(-0x80000000)
            mx_b = jnp.max(mk) ^ sbit
            tau_b = jnp.max(tau_est) ^ sbit
            diff = mx_b ^ tau_b

            def hbody(j, hb):
                return jnp.where(
                    lax.shift_right_logical(diff, j) != 0, j, hb)
            hb = lax.fori_loop(0, 32, hbody, jnp.int32(-1))
            sh = jnp.int32(31) - jnp.clip(hb, 0, 31)
            low = lax.shift_right_logical(jnp.int32(-1), sh)
            p0 = mx_b & ~jnp.where(hb < 0, jnp.int32(0), low)
            return search(nv, ptr, jnp.int32(K),
                          hb, p0.astype(jnp.uint32), False)

        def fallback_path(n, ptr):
            @plsc.parallel_loop(0, NBINS, unroll=8)
            def _(b):
                hist[b] = jnp.zeros((L,), jnp.int32)

            @plsc.parallel_loop(0, NVREG, unroll=8)
            def _(i):
                k = plsc.bitcast(row_v[pl.ds(i * L, L)], jnp.int32)
                byte = ((jnp.right_shift(k, 24) & jnp.int32(0xFF))
                        ^ jnp.int32(0x80))
                plsc.addupdate_scatter(hist, [byte, lane], ones)

            @plsc.parallel_loop(
                0, NBINS, unroll=8,
                carry=(jnp.int32(0), jnp.int32(-1), jnp.int32(0)))
            def sb_out(j, carry):
                acc, bsel, above = carry
                b = jnp.int32(NBINS - 1) - j
                cnt = jnp.sum(hist[b])
                found_now = jnp.logical_and(bsel < 0, acc + cnt >= K)
                bsel = jnp.where(found_now, b, bsel)
                above = jnp.where(found_now, acc, above)
                return (acc + cnt, bsel, above)
            _, b1, above = sb_out
            krem = jnp.int32(K) - above
            b1 = b1 ^ jnp.int32(0x80)

            @plsc.parallel_loop(0, NVREG, unroll=4,
                                carry=jnp.zeros((L,), jnp.int32))
            def ptr2(i, ptr2):
                kv = plsc.bitcast(row_v[pl.ds(i * L, L)], jnp.int32)
                keep = (jnp.right_shift(kv, 24) & jnp.int32(0xFF)) == b1
                ki = keep.astype(jnp.int32)
                idx = ptr2 + plsc.cumsum(ki) - ki
                plsc.store_scatter(cand_v, [idx],
                                   kv ^ jnp.int32(-0x80000000), mask=keep)
                return ptr2 + plsc.all_reduce_population_count(keep)
            n2 = jnp.max(ptr2)
            nv2 = (n2 + (L - 1)) // L
            p0 = jnp.left_shift(b1 ^ jnp.int32(0x80), 24)
            return search(nv2, ptr2, krem,
                          jnp.int32(23), p0.astype(jnp.uint32), True)

        with jax.named_scope("sel"):
            good = jnp.logical_and(n >= K, n <= C - L)
            tau = lax.cond(good, good_path, fallback_path, n, ptr)

        out_h = []
        with jax.named_scope("p3"):
            for ch in range(NCH):
                if ch == 2:
                    out_h[0].wait()
                    row_n = jnp.minimum(row + 1, jnp.int32(BATCH - 1))
                    pltpu.async_copy(
                        x_hbm.at[row_n, pl.ds(0, CH)],
                        row_v.at[pl.ds(0, CH)],
                        sems_in[0])
                base = ch * CHV

                @plsc.parallel_loop(0, CHV, unroll=8)
                def _(i, base=base):
                    sl = pl.ds((base + i) * L, L)
                    kv = plsc.bitcast(row_v[sl], jnp.int32)
                    y = _float_from_key(kv)
                    row_v[sl] = jnp.where(kv >= tau, y, jnp.float32(0.0))

                out_h.append(pltpu.async_copy(
                    row_v.at[pl.ds(ch * CH, CH)],
                    out_hbm.at[row, pl.ds(ch * CH, CH)],
                    sems_out[ch]))
            for h in out_h[1:]:
                h.wait()
        return 0

    lax.fori_loop(0, ROWS_PER_W, row_fn, 0)
    pltpu.make_async_copy(
        x_hbm.at[0, pl.ds(0, CH)],
        row_v.at[pl.ds(0, CH)],
        sems_in[0]).wait()


@functools.partial(jax.jit)
def _sc_kfeature(x, w):
    mesh = plsc.VectorSubcoreMesh(core_axis_name="c", subcore_axis_name="s")
    f = functools.partial(
        pl.kernel,
        mesh=mesh,
        compiler_params=pltpu.CompilerParams(needs_layout_passes=False),
        out_type=jax.ShapeDtypeStruct((BATCH, C), jnp.float32),
        scratch_types=[
            pltpu.VMEM((C,), jnp.float32),
            pltpu.VMEM((C,), jnp.float32),
            pltpu.VMEM((C,), jnp.int32),
            pltpu.VMEM((NBINS, L), jnp.int32),
            pltpu.SemaphoreType.DMA,
            pltpu.SemaphoreType.DMA,
            pltpu.SemaphoreType.DMA,
            pltpu.SemaphoreType.DMA,
            pltpu.SemaphoreType.DMA,
            pltpu.SemaphoreType.DMA,
            pltpu.SemaphoreType.DMA,
        ],
    )(_sc_body)
    return f(x, w)


def kernel(x, w):
    return _sc_kfeature(x, w)
